# baseline (speedup 1.0000x reference)
import jax
import jax.numpy as jnp
from jax.experimental import pallas as pl
from jax.experimental.pallas import tpu as pltpu

_R = 1024
_C = 26
_SIZE = 1000
_B = 32
_NBUF = 4
_NCHUNK = _R // _B


def _body(idx_ref, out_ref, *scratch):
    bufs = scratch[:_NBUF]
    sems = scratch[_NBUF:]
    for c in range(_NCHUNK):
        b = c % _NBUF
        if c >= _NBUF:
            prev = c - _NBUF
            pltpu.make_async_copy(
                bufs[b], out_ref.at[pl.ds(prev * _B, _B)], sems[b]
            ).wait()
        idx = idx_ref[pl.ds(c * _B, _B)].astype(jnp.int32)
        classes = jax.lax.broadcasted_iota(jnp.int32, (_B, _C, _SIZE), 2)
        bufs[b][...] = (classes == idx).astype(jnp.float32)
        pltpu.make_async_copy(
            bufs[b], out_ref.at[pl.ds(c * _B, _B)], sems[b]
        ).start()
    for c in range(_NCHUNK - _NBUF, _NCHUNK):
        b = c % _NBUF
        pltpu.make_async_copy(
            bufs[b], out_ref.at[pl.ds(c * _B, _B)], sems[b]
        ).wait()


def kernel(x, size):
    del size
    idx = x.reshape(_R, _C, 1)
    return pl.pallas_call(
        _body,
        in_specs=[pl.BlockSpec(memory_space=pltpu.MemorySpace.VMEM)],
        out_specs=pl.BlockSpec(memory_space=pl.ANY),
        out_shape=jax.ShapeDtypeStruct((_R, _C, _SIZE), jnp.float32),
        scratch_shapes=(
            [pltpu.VMEM((_B, _C, _SIZE), jnp.float32)] * _NBUF
            + [pltpu.SemaphoreType.DMA] * _NBUF
        ),
    )(idx)


# speedup vs baseline: 5.2483x; 5.2483x over previous
import jax
import jax.numpy as jnp
from jax.experimental import pallas as pl

_R = 1024
_C = 26
_SIZE = 1000


def _onehot_block(idx_ref, out_ref):
    idx = idx_ref[...]
    classes = jax.lax.broadcasted_iota(jnp.int32, (1, _SIZE, _R), 1)
    out_ref[...] = (classes == idx).astype(jnp.float32)


def kernel(x, size):
    del size
    idx_t = x.astype(jnp.int32).T.reshape(_C, 1, _R)
    out_t = pl.pallas_call(
        _onehot_block,
        grid=(_C,),
        in_specs=[pl.BlockSpec((1, 1, _R), lambda i: (i, 0, 0))],
        out_specs=pl.BlockSpec((1, _SIZE, _R), lambda i: (i, 0, 0)),
        out_shape=jax.ShapeDtypeStruct((_C, _SIZE, _R), jnp.float32),
    )(idx_t)
    return out_t.transpose(2, 0, 1)
